# Initial kernel scaffold; baseline (speedup 1.0000x reference)
import jax
import jax.numpy as jnp
from jax.experimental import pallas as pl


def kernel(ensemble, x, edge_index, edge_attr, deepset, Wd, bd, convs, Wa, ba):
    raise NotImplementedError("write your pallas kernel here")


# speedup vs baseline: 2.2806x; 2.2806x over previous
import functools

import jax
import jax.numpy as jnp
from jax import lax
from jax.experimental import pallas as pl
from jax.experimental.pallas import tpu as pltpu
from jax.experimental.pallas import tpu_sc as plsc

N = 10000
E = 320000
D_IN = 128
H = 64
ENS = 10

NB = 200
GRID_PRE = N // NB

C = 128
NW = 32
CPW = (E + NW * C - 1) // (NW * C)
E_PAD = NW * C * CPW
SUB_ROWS = 632
N_PAD = SUB_ROWS * 16


def _pre_body(ens_ref, x_ref, w1, b1, w2, b2, w3, b3, w4, b4, wdx, wde, bd,
              out_ref):
    ens = ens_ref[...].reshape(NB * ENS, D_IN)
    phi = jnp.maximum(jnp.dot(ens, w1[...], preferred_element_type=jnp.float32)
                      + b1[...], 0.0)
    phi = jnp.dot(phi, w2[...], preferred_element_type=jnp.float32) + b2[...]
    agg = phi.reshape(NB, ENS, H).sum(axis=1)
    emb = jnp.maximum(jnp.dot(agg, w3[...], preferred_element_type=jnp.float32)
                      + b3[...], 0.0)
    emb = jnp.dot(emb, w4[...], preferred_element_type=jnp.float32) + b4[...]
    nf = (jnp.dot(x_ref[...], wdx[...], preferred_element_type=jnp.float32)
          + jnp.dot(emb, wde[...], preferred_element_type=jnp.float32)
          + bd[...])
    out_ref[...] = nf


def _pre(ensemble, x, ds, Wd, bd):
    full = lambda shape: pl.BlockSpec(shape, lambda i: (0,) * len(shape))
    return pl.pallas_call(
        _pre_body,
        grid=(GRID_PRE,),
        in_specs=[
            pl.BlockSpec((NB, ENS, D_IN), lambda i: (i, 0, 0)),
            pl.BlockSpec((NB, D_IN), lambda i: (i, 0)),
            full((D_IN, H)), full((1, H)),
            full((H, H)), full((1, H)),
            full((H, H)), full((1, H)),
            full((H, H)), full((1, H)),
            full((D_IN, H)), full((H, H)), full((1, H)),
        ],
        out_specs=pl.BlockSpec((NB, H), lambda i: (i, 0)),
        out_shape=jax.ShapeDtypeStruct((N, H), jnp.float32),
    )(ensemble, x,
      ds['W1'], ds['b1'].reshape(1, H),
      ds['W2'], ds['b2'].reshape(1, H),
      ds['W3'], ds['b3'].reshape(1, H),
      ds['W4'], ds['b4'].reshape(1, H),
      Wd[:D_IN], Wd[D_IN:], bd.reshape(1, H))


def _mp_body(h_hbm, src_hbm, dst_hbm, ea_hbm, wb_hbm, z_hbm, out_hbm,
             srcv, dstv, eav, rowsv, wbv, acc, sem):
    c = lax.axis_index("c")
    s = lax.axis_index("s")
    wid = s * 2 + c

    pltpu.sync_copy(z_hbm.at[pl.ds(s * SUB_ROWS, SUB_ROWS)],
                    acc.at[pl.ds(s * SUB_ROWS, SUB_ROWS)])
    pltpu.sync_copy(wb_hbm, wbv)
    plsc.subcore_barrier()

    @pl.loop(0, CPW)
    def _chunk(j):
        chunk = wid * CPW + j
        base = chunk * C
        pltpu.sync_copy(src_hbm.at[pl.ds(base, C)], srcv)
        pltpu.sync_copy(ea_hbm.at[pl.ds(base, C)], eav)
        pltpu.sync_copy(dst_hbm.at[chunk], dstv)
        pltpu.async_copy(h_hbm.at[srcv], rowsv, sem).wait()

        @pl.loop(0, C // 16)
        def _blk(jj):
            a_vec = eav[pl.ds(jj * 16, 16)]
            for ii in range(16):
                i = jj * 16 + ii
                a = a_vec[ii]
                for g in range(4):
                    sl = pl.ds(g * 16, 16)
                    e = wbv[0, sl] * a + wbv[1, sl]
                    rowsv[i, sl] = jnp.maximum(rowsv[i, sl] + e, 0.0)

        pltpu.sync_copy(rowsv, acc.at[dstv.at[0]], add=True)

    plsc.subcore_barrier()
    pltpu.sync_copy(acc.at[pl.ds(s * SUB_ROWS, SUB_ROWS)],
                    out_hbm.at[c, pl.ds(s * SUB_ROWS, SUB_ROWS)])


@functools.lru_cache(maxsize=None)
def _mp_call():
    return pl.kernel(
        _mp_body,
        mesh=plsc.VectorSubcoreMesh(core_axis_name="c", subcore_axis_name="s"),
        out_type=jax.ShapeDtypeStruct((2, N_PAD, H), jnp.float32),
        scratch_types=[
            pltpu.VMEM((C,), jnp.int32),
            pltpu.VMEM((1, C), jnp.int32),
            pltpu.VMEM((C,), jnp.float32),
            pltpu.VMEM((C, H), jnp.float32),
            pltpu.VMEM((2, H), jnp.float32),
            pltpu.VMEM_SHARED((N_PAD, H), jnp.float32),
            pltpu.SemaphoreType.DMA,
        ],
        compiler_params=pltpu.CompilerParams(use_tc_tiling_on_sc=False),
    )


def _mlp_body(first, last, h_ref, a0_ref, a1_ref, eps_ref, wm1, bm1, gm, bt,
              wm2, bm2, wa, ba, out_ref):
    h = h_ref[...]
    z = h * (1.0 + eps_ref[0, 0]) + a0_ref[...] + a1_ref[...]
    y = jnp.dot(z, wm1[...], preferred_element_type=jnp.float32) + bm1[...]
    mean = jnp.mean(y, axis=0, keepdims=True)
    var = jnp.mean(jnp.square(y - mean), axis=0, keepdims=True)
    y = (y - mean) / jnp.sqrt(var + 1e-5) * gm[...] + bt[...]
    y = jnp.maximum(y, 0.0)
    cc = jnp.dot(y, wm2[...], preferred_element_type=jnp.float32) + bm2[...]
    hn = jnp.maximum(cc, 0.0) if first else h + jnp.maximum(cc, 0.0)
    if last:
        o = jnp.dot(hn, wa[...], preferred_element_type=jnp.float32) + ba[...]
        sp = jnp.maximum(o, 0.0) + jnp.log1p(jnp.exp(-jnp.abs(o)))
        col = lax.broadcasted_iota(jnp.int32, o.shape, 1)
        out_ref[...] = jnp.where(col == 0, o, sp)
    else:
        out_ref[...] = hn


def _mlp(first, last, h, a0, a1, p, Wa, ba):
    odim = 2 if last else H
    body = functools.partial(_mlp_body, first, last)
    return pl.pallas_call(
        body,
        out_shape=jax.ShapeDtypeStruct((N, odim), jnp.float32),
    )(h, a0, a1, p['eps'].reshape(1, 1),
      p['Wm1'], p['bm1'].reshape(1, H),
      p['gamma'].reshape(1, H), p['beta'].reshape(1, H),
      p['Wm2'], p['bm2'].reshape(1, H),
      Wa, ba.reshape(1, 2))


def kernel(ensemble, x, edge_index, edge_attr, deepset, Wd, bd, convs, Wa, ba):
    nf = _pre(ensemble, x, deepset, Wd, bd)

    src = jnp.concatenate(
        [edge_index[0], jnp.zeros((E_PAD - E,), jnp.int32)])
    dst = jnp.concatenate(
        [edge_index[1], jnp.full((E_PAD - E,), N_PAD - 1, jnp.int32)])
    dst3 = dst.reshape(NW * CPW, 1, C)
    ea = jnp.concatenate(
        [edge_attr[:, 0], jnp.zeros((E_PAD - E,), jnp.float32)])
    zeros = jnp.zeros((N_PAD, H), jnp.float32)

    h = nf
    for i, p in enumerate(convs):
        wb = jnp.stack([p['We'][0], p['be']])
        out = _mp_call()(h, src, dst3, ea, wb, zeros)
        h = _mlp(i == 0, i == len(convs) - 1,
                 h, out[0, :N], out[1, :N], p, Wa, ba)
    return h


# speedup vs baseline: 4.7722x; 2.0925x over previous
import functools

import jax
import jax.numpy as jnp
from jax import lax
from jax.experimental import pallas as pl
from jax.experimental.pallas import tpu as pltpu
from jax.experimental.pallas import tpu_sc as plsc

N = 10000
E = 320000
D_IN = 128
H = 64
ENS = 10

NB = 200
GRID_PRE = N // NB

C = 128
NW = 32
CPW = 80
E_PAD = NW * C * CPW
NBUF = 4
LOOK = 2
SUB_ROWS = 632
N_PAD = SUB_ROWS * 16


def _pre_body(ens_ref, x_ref, w1, b1, w2, b2, w3, b3, w4, b4, wdx, wde, bd,
              out_ref):
    ens = ens_ref[...].reshape(NB * ENS, D_IN)
    phi = jnp.maximum(jnp.dot(ens, w1[...], preferred_element_type=jnp.float32)
                      + b1[...], 0.0)
    phi = jnp.dot(phi, w2[...], preferred_element_type=jnp.float32) + b2[...]
    agg = phi.reshape(NB, ENS, H).sum(axis=1)
    emb = jnp.maximum(jnp.dot(agg, w3[...], preferred_element_type=jnp.float32)
                      + b3[...], 0.0)
    emb = jnp.dot(emb, w4[...], preferred_element_type=jnp.float32) + b4[...]
    nf = (jnp.dot(x_ref[...], wdx[...], preferred_element_type=jnp.float32)
          + jnp.dot(emb, wde[...], preferred_element_type=jnp.float32)
          + bd[...])
    out_ref[...] = nf


def _pre(ensemble, x, ds, Wd, bd):
    full = lambda shape: pl.BlockSpec(shape, lambda i: (0,) * len(shape))
    return pl.pallas_call(
        _pre_body,
        grid=(GRID_PRE,),
        in_specs=[
            pl.BlockSpec((NB, ENS, D_IN), lambda i: (i, 0, 0)),
            pl.BlockSpec((NB, D_IN), lambda i: (i, 0)),
            full((D_IN, H)), full((1, H)),
            full((H, H)), full((1, H)),
            full((H, H)), full((1, H)),
            full((H, H)), full((1, H)),
            full((D_IN, H)), full((H, H)), full((1, H)),
        ],
        out_specs=pl.BlockSpec((NB, H), lambda i: (i, 0)),
        out_shape=jax.ShapeDtypeStruct((N, H), jnp.float32),
    )(ensemble, x,
      ds['W1'], ds['b1'].reshape(1, H),
      ds['W2'], ds['b2'].reshape(1, H),
      ds['W3'], ds['b3'].reshape(1, H),
      ds['W4'], ds['b4'].reshape(1, H),
      Wd[:D_IN], Wd[D_IN:], bd.reshape(1, H))


def _mp_body(h_hbm, src_hbm, dst_hbm, ea_hbm, wb_hbm, z_hbm, out_hbm,
             srcv, dstv, eav, rowsv, wbv, acc, gsem, ssem):
    c = lax.axis_index("c")
    s = lax.axis_index("s")
    wid = s * 2 + c

    pltpu.sync_copy(src_hbm.at[wid], srcv)
    pltpu.sync_copy(dst_hbm.at[wid], dstv)
    pltpu.sync_copy(ea_hbm.at[wid], eav)
    pltpu.sync_copy(wb_hbm, wbv)

    def fire_gather(j, b):
        pltpu.async_copy(h_hbm.at[srcv.at[j]], rowsv.at[b], gsem.at[b])

    def wait_gather(j, b):
        pltpu.make_async_copy(h_hbm.at[srcv.at[j]], rowsv.at[b],
                              gsem.at[b]).wait()

    def fire_scatter(j, b):
        pltpu.async_copy(rowsv.at[b], acc.at[dstv.at[j]], ssem.at[b],
                         add=True)

    def wait_scatter(j, b):
        pltpu.make_async_copy(rowsv.at[b], acc.at[dstv.at[j]],
                              ssem.at[b]).wait()

    for b in range(LOOK):
        fire_gather(b, b)

    pltpu.sync_copy(z_hbm.at[pl.ds(s * SUB_ROWS, SUB_ROWS)],
                    acc.at[pl.ds(s * SUB_ROWS, SUB_ROWS)])
    plsc.subcore_barrier()

    we = [wbv[0, pl.ds(g * 16, 16)] for g in range(4)]
    be = [wbv[1, pl.ds(g * 16, 16)] for g in range(4)]

    @pl.loop(0, CPW // NBUF)
    def _outer(jo):
        for b in range(NBUF):
            j = jo * NBUF + b
            bn = (b + LOOK) % NBUF

            @pl.when(j + LOOK < CPW)
            def _fire():
                @pl.when(j + LOOK >= NBUF)
                def _drain():
                    wait_scatter(j + LOOK - NBUF, bn)
                fire_gather(j + LOOK, bn)

            wait_gather(j, b)

            @pl.loop(0, C // 16)
            def _blk(jj):
                a_vec = eav[j, pl.ds(jj * 16, 16)]
                for ii in range(16):
                    i = jj * 16 + ii
                    a = a_vec[ii]
                    for g in range(4):
                        sl = pl.ds(g * 16, 16)
                        e = we[g] * a + be[g]
                        rowsv[b, i, sl] = jnp.maximum(rowsv[b, i, sl] + e,
                                                      0.0)

            fire_scatter(j, b)

    for b in range(NBUF):
        wait_scatter(CPW - NBUF + b, b)

    plsc.subcore_barrier()
    pltpu.sync_copy(acc.at[pl.ds(s * SUB_ROWS, SUB_ROWS)],
                    out_hbm.at[c, pl.ds(s * SUB_ROWS, SUB_ROWS)])


@functools.lru_cache(maxsize=None)
def _mp_call():
    return pl.kernel(
        _mp_body,
        mesh=plsc.VectorSubcoreMesh(core_axis_name="c", subcore_axis_name="s"),
        out_type=jax.ShapeDtypeStruct((2, N_PAD, H), jnp.float32),
        scratch_types=[
            pltpu.VMEM((CPW, C), jnp.int32),
            pltpu.VMEM((CPW, C), jnp.int32),
            pltpu.VMEM((CPW, C), jnp.float32),
            pltpu.VMEM((NBUF, C, H), jnp.float32),
            pltpu.VMEM((2, H), jnp.float32),
            pltpu.VMEM_SHARED((N_PAD, H), jnp.float32),
            pltpu.SemaphoreType.DMA((NBUF,)),
            pltpu.SemaphoreType.DMA((NBUF,)),
        ],
        compiler_params=pltpu.CompilerParams(use_tc_tiling_on_sc=False),
    )


def _mlp_body(first, last, h_ref, a0_ref, a1_ref, eps_ref, wm1, bm1, gm, bt,
              wm2, bm2, wa, ba, out_ref):
    h = h_ref[...]
    z = h * (1.0 + eps_ref[0, 0]) + a0_ref[...] + a1_ref[...]
    y = jnp.dot(z, wm1[...], preferred_element_type=jnp.float32) + bm1[...]
    mean = jnp.mean(y, axis=0, keepdims=True)
    var = jnp.mean(jnp.square(y - mean), axis=0, keepdims=True)
    y = (y - mean) / jnp.sqrt(var + 1e-5) * gm[...] + bt[...]
    y = jnp.maximum(y, 0.0)
    cc = jnp.dot(y, wm2[...], preferred_element_type=jnp.float32) + bm2[...]
    hn = jnp.maximum(cc, 0.0) if first else h + jnp.maximum(cc, 0.0)
    if last:
        o = jnp.dot(hn, wa[...], preferred_element_type=jnp.float32) + ba[...]
        sp = jnp.maximum(o, 0.0) + jnp.log1p(jnp.exp(-jnp.abs(o)))
        col = lax.broadcasted_iota(jnp.int32, o.shape, 1)
        out_ref[...] = jnp.where(col == 0, o, sp)
    else:
        out_ref[...] = hn


def _mlp(first, last, h, a0, a1, p, Wa, ba):
    odim = 2 if last else H
    body = functools.partial(_mlp_body, first, last)
    return pl.pallas_call(
        body,
        out_shape=jax.ShapeDtypeStruct((N, odim), jnp.float32),
    )(h, a0, a1, p['eps'].reshape(1, 1),
      p['Wm1'], p['bm1'].reshape(1, H),
      p['gamma'].reshape(1, H), p['beta'].reshape(1, H),
      p['Wm2'], p['bm2'].reshape(1, H),
      Wa, ba.reshape(1, 2))


def kernel(ensemble, x, edge_index, edge_attr, deepset, Wd, bd, convs, Wa, ba):
    nf = _pre(ensemble, x, deepset, Wd, bd)

    src = jnp.concatenate(
        [edge_index[0], jnp.zeros((E_PAD - E,), jnp.int32)]
    ).reshape(NW, CPW, C)
    dst3 = jnp.concatenate(
        [edge_index[1], jnp.full((E_PAD - E,), N_PAD - 1, jnp.int32)]
    ).reshape(NW, CPW, C)
    ea = jnp.concatenate(
        [edge_attr[:, 0], jnp.zeros((E_PAD - E,), jnp.float32)]
    ).reshape(NW, CPW, C)
    zeros = jnp.zeros((N_PAD, H), jnp.float32)

    h = nf
    for i, p in enumerate(convs):
        wb = jnp.stack([p['We'][0], p['be']])
        out = _mp_call()(h, src, dst3, ea, wb, zeros)
        h = _mlp(i == 0, i == len(convs) - 1,
                 h, out[0, :N], out[1, :N], p, Wa, ba)
    return h


# speedup vs baseline: 4.8754x; 1.0216x over previous
import functools

import jax
import jax.numpy as jnp
from jax import lax
from jax.experimental import pallas as pl
from jax.experimental.pallas import tpu as pltpu
from jax.experimental.pallas import tpu_sc as plsc

N = 10000
E = 320000
D_IN = 128
H = 64
ENS = 10

NB = 200
GRID_PRE = N // NB

C = 128
NW = 32
TOTCH = 2560
E_PAD = TOTCH * C
CPW0 = 120
CPW1 = 40
CPWMAX = 120
NBUF = 4
LOOK = 2
SUB_ROWS = 632
N_PAD = SUB_ROWS * 16


def _pre_body(ens_ref, x_ref, w1, b1, w2, b2, w3, b3, w4, b4, wdx, wde, bd,
              out_ref):
    ens = ens_ref[...].reshape(NB * ENS, D_IN)
    phi = jnp.maximum(jnp.dot(ens, w1[...], preferred_element_type=jnp.float32)
                      + b1[...], 0.0)
    phi = jnp.dot(phi, w2[...], preferred_element_type=jnp.float32) + b2[...]
    agg = phi.reshape(NB, ENS, H).sum(axis=1)
    emb = jnp.maximum(jnp.dot(agg, w3[...], preferred_element_type=jnp.float32)
                      + b3[...], 0.0)
    emb = jnp.dot(emb, w4[...], preferred_element_type=jnp.float32) + b4[...]
    nf = (jnp.dot(x_ref[...], wdx[...], preferred_element_type=jnp.float32)
          + jnp.dot(emb, wde[...], preferred_element_type=jnp.float32)
          + bd[...])
    out_ref[...] = nf


def _pre(ensemble, x, ds, Wd, bd):
    full = lambda shape: pl.BlockSpec(shape, lambda i: (0,) * len(shape))
    return pl.pallas_call(
        _pre_body,
        grid=(GRID_PRE,),
        in_specs=[
            pl.BlockSpec((NB, ENS, D_IN), lambda i: (i, 0, 0)),
            pl.BlockSpec((NB, D_IN), lambda i: (i, 0)),
            full((D_IN, H)), full((1, H)),
            full((H, H)), full((1, H)),
            full((H, H)), full((1, H)),
            full((H, H)), full((1, H)),
            full((D_IN, H)), full((H, H)), full((1, H)),
        ],
        out_specs=pl.BlockSpec((NB, H), lambda i: (i, 0)),
        out_shape=jax.ShapeDtypeStruct((N, H), jnp.float32),
    )(ensemble, x,
      ds['W1'], ds['b1'].reshape(1, H),
      ds['W2'], ds['b2'].reshape(1, H),
      ds['W3'], ds['b3'].reshape(1, H),
      ds['W4'], ds['b4'].reshape(1, H),
      Wd[:D_IN], Wd[D_IN:], bd.reshape(1, H))


def _mp_body(h_hbm, src_hbm, dst_hbm, ea_hbm, wb_hbm, z_hbm, out_hbm,
             srcv, dstv, eav, rowsv, wbv, acc, gsem, ssem):
    c = lax.axis_index("c")
    s = lax.axis_index("s")

    pltpu.sync_copy(z_hbm.at[pl.ds(s * SUB_ROWS, SUB_ROWS)],
                    acc.at[pl.ds(s * SUB_ROWS, SUB_ROWS)])
    pltpu.sync_copy(wb_hbm, wbv)

    def fire_gather(j, b):
        pltpu.async_copy(h_hbm.at[srcv.at[j]], rowsv.at[b], gsem.at[b])

    def wait_gather(j, b):
        pltpu.make_async_copy(h_hbm.at[srcv.at[j]], rowsv.at[b],
                              gsem.at[b]).wait()

    def fire_scatter(j, b):
        pltpu.async_copy(rowsv.at[b], acc.at[dstv.at[j]], ssem.at[b],
                         add=True)

    def wait_scatter(j, b):
        pltpu.make_async_copy(rowsv.at[b], acc.at[dstv.at[j]],
                              ssem.at[b]).wait()

    we = [wbv[0, pl.ds(g * 16, 16)] for g in range(4)]
    be = [wbv[1, pl.ds(g * 16, 16)] for g in range(4)]

    def run(start, cpw):
        pltpu.sync_copy(src_hbm.at[pl.ds(start, cpw)],
                        srcv.at[pl.ds(0, cpw)])
        pltpu.sync_copy(dst_hbm.at[pl.ds(start, cpw)],
                        dstv.at[pl.ds(0, cpw)])
        pltpu.sync_copy(ea_hbm.at[pl.ds(start, cpw)],
                        eav.at[pl.ds(0, cpw)])

        for b in range(LOOK):
            fire_gather(b, b)

        plsc.subcore_barrier()

        @pl.loop(0, cpw // NBUF)
        def _outer(jo):
            for b in range(NBUF):
                j = jo * NBUF + b
                bn = (b + LOOK) % NBUF

                @pl.when(j + LOOK < cpw)
                def _fire():
                    @pl.when(j + LOOK >= NBUF)
                    def _drain():
                        wait_scatter(j + LOOK - NBUF, bn)
                    fire_gather(j + LOOK, bn)

                wait_gather(j, b)

                @pl.loop(0, C // 16)
                def _blk(jj):
                    a_vec = eav[j, pl.ds(jj * 16, 16)]
                    for ii in range(16):
                        i = jj * 16 + ii
                        a = a_vec[ii]
                        for g in range(4):
                            sl = pl.ds(g * 16, 16)
                            e = we[g] * a + be[g]
                            rowsv[b, i, sl] = jnp.maximum(
                                rowsv[b, i, sl] + e, 0.0)

                fire_scatter(j, b)

        for b in range(NBUF):
            wait_scatter(cpw - NBUF + b, b)

    @pl.when(c == 0)
    def _core0():
        run(s * CPW0, CPW0)

    @pl.when(c == 1)
    def _core1():
        run(16 * CPW0 + s * CPW1, CPW1)

    plsc.subcore_barrier()
    pltpu.sync_copy(acc.at[pl.ds(s * SUB_ROWS, SUB_ROWS)],
                    out_hbm.at[c, pl.ds(s * SUB_ROWS, SUB_ROWS)])


@functools.lru_cache(maxsize=None)
def _mp_call():
    return pl.kernel(
        _mp_body,
        mesh=plsc.VectorSubcoreMesh(core_axis_name="c", subcore_axis_name="s"),
        out_type=jax.ShapeDtypeStruct((2, N_PAD, H), jnp.float32),
        scratch_types=[
            pltpu.VMEM((CPWMAX, C), jnp.int32),
            pltpu.VMEM((CPWMAX, C), jnp.int32),
            pltpu.VMEM((CPWMAX, C), jnp.float32),
            pltpu.VMEM((NBUF, C, H), jnp.float32),
            pltpu.VMEM((2, H), jnp.float32),
            pltpu.VMEM_SHARED((N_PAD, H), jnp.float32),
            pltpu.SemaphoreType.DMA((NBUF,)),
            pltpu.SemaphoreType.DMA((NBUF,)),
        ],
        compiler_params=pltpu.CompilerParams(use_tc_tiling_on_sc=False),
    )


def _mlp_body(first, last, h_ref, a0_ref, a1_ref, eps_ref, wm1, bm1, gm, bt,
              wm2, bm2, wa, ba, out_ref):
    h = h_ref[...]
    z = h * (1.0 + eps_ref[0, 0]) + a0_ref[...] + a1_ref[...]
    y = jnp.dot(z, wm1[...], preferred_element_type=jnp.float32) + bm1[...]
    mean = jnp.mean(y, axis=0, keepdims=True)
    var = jnp.mean(jnp.square(y - mean), axis=0, keepdims=True)
    y = (y - mean) / jnp.sqrt(var + 1e-5) * gm[...] + bt[...]
    y = jnp.maximum(y, 0.0)
    cc = jnp.dot(y, wm2[...], preferred_element_type=jnp.float32) + bm2[...]
    hn = jnp.maximum(cc, 0.0) if first else h + jnp.maximum(cc, 0.0)
    if last:
        o = jnp.dot(hn, wa[...], preferred_element_type=jnp.float32) + ba[...]
        sp = jnp.maximum(o, 0.0) + jnp.log1p(jnp.exp(-jnp.abs(o)))
        col = lax.broadcasted_iota(jnp.int32, o.shape, 1)
        out_ref[...] = jnp.where(col == 0, o, sp)
    else:
        out_ref[...] = hn


def _mlp(first, last, h, a0, a1, p, Wa, ba):
    odim = 2 if last else H
    body = functools.partial(_mlp_body, first, last)
    return pl.pallas_call(
        body,
        out_shape=jax.ShapeDtypeStruct((N, odim), jnp.float32),
    )(h, a0, a1, p['eps'].reshape(1, 1),
      p['Wm1'], p['bm1'].reshape(1, H),
      p['gamma'].reshape(1, H), p['beta'].reshape(1, H),
      p['Wm2'], p['bm2'].reshape(1, H),
      Wa, ba.reshape(1, 2))


def kernel(ensemble, x, edge_index, edge_attr, deepset, Wd, bd, convs, Wa, ba):
    nf = _pre(ensemble, x, deepset, Wd, bd)

    src = jnp.concatenate(
        [edge_index[0], jnp.zeros((E_PAD - E,), jnp.int32)]
    ).reshape(TOTCH, C)
    dst3 = jnp.concatenate(
        [edge_index[1], jnp.full((E_PAD - E,), N_PAD - 1, jnp.int32)]
    ).reshape(TOTCH, C)
    ea = jnp.concatenate(
        [edge_attr[:, 0], jnp.zeros((E_PAD - E,), jnp.float32)]
    ).reshape(TOTCH, C)
    zeros = jnp.zeros((N_PAD, H), jnp.float32)

    h = nf
    for i, p in enumerate(convs):
        wb = jnp.stack([p['We'][0], p['be']])
        out = _mp_call()(h, src, dst3, ea, wb, zeros)
        h = _mlp(i == 0, i == len(convs) - 1,
                 h, out[0, :N], out[1, :N], p, Wa, ba)
    return h


# speedup vs baseline: 4.8839x; 1.0017x over previous
import functools

import jax
import jax.numpy as jnp
from jax import lax
from jax.experimental import pallas as pl
from jax.experimental.pallas import tpu as pltpu
from jax.experimental.pallas import tpu_sc as plsc

N = 10000
E = 320000
D_IN = 128
H = 64
ENS = 10

NB = 200
GRID_PRE = N // NB

C = 128
NW = 32
TOTCH = 2560
E_PAD = TOTCH * C
CPW0 = 120
CPW1 = 40
CPWMAX = 120
NBUF = 4
LOOK = 2
SUB_ROWS = 632
N_PAD = SUB_ROWS * 16


def _pre_body(ens_ref, x_ref, w1, b1, w2, b2, w3, b3, w4, b4, wdx, wde, bd,
              out_ref):
    ens = ens_ref[...].reshape(NB * ENS, D_IN)
    phi = jnp.maximum(jnp.dot(ens, w1[...], preferred_element_type=jnp.float32)
                      + b1[...], 0.0)
    phi = jnp.dot(phi, w2[...], preferred_element_type=jnp.float32) + b2[...]
    agg = phi.reshape(NB, ENS, H).sum(axis=1)
    emb = jnp.maximum(jnp.dot(agg, w3[...], preferred_element_type=jnp.float32)
                      + b3[...], 0.0)
    emb = jnp.dot(emb, w4[...], preferred_element_type=jnp.float32) + b4[...]
    nf = (jnp.dot(x_ref[...], wdx[...], preferred_element_type=jnp.float32)
          + jnp.dot(emb, wde[...], preferred_element_type=jnp.float32)
          + bd[...])
    out_ref[...] = nf


def _pre(ensemble, x, ds, Wd, bd):
    full = lambda shape: pl.BlockSpec(shape, lambda i: (0,) * len(shape))
    return pl.pallas_call(
        _pre_body,
        grid=(GRID_PRE,),
        in_specs=[
            pl.BlockSpec((NB, ENS, D_IN), lambda i: (i, 0, 0)),
            pl.BlockSpec((NB, D_IN), lambda i: (i, 0)),
            full((D_IN, H)), full((1, H)),
            full((H, H)), full((1, H)),
            full((H, H)), full((1, H)),
            full((H, H)), full((1, H)),
            full((D_IN, H)), full((H, H)), full((1, H)),
        ],
        out_specs=pl.BlockSpec((NB, H), lambda i: (i, 0)),
        out_shape=jax.ShapeDtypeStruct((N, H), jnp.float32),
    )(ensemble, x,
      ds['W1'], ds['b1'].reshape(1, H),
      ds['W2'], ds['b2'].reshape(1, H),
      ds['W3'], ds['b3'].reshape(1, H),
      ds['W4'], ds['b4'].reshape(1, H),
      Wd[:D_IN], Wd[D_IN:], bd.reshape(1, H))


def _mp_body(h_hbm, src_hbm, dst_hbm, ea_hbm, wb_hbm, z_hbm, out_hbm,
             srcv, dstv, eav, rowsv, wbv, acc, gsem, ssem):
    c = lax.axis_index("c")
    s = lax.axis_index("s")

    with jax.named_scope("zero_acc"):
        pltpu.sync_copy(z_hbm.at[pl.ds(s * SUB_ROWS, SUB_ROWS)],
                        acc.at[pl.ds(s * SUB_ROWS, SUB_ROWS)])
        pltpu.sync_copy(wb_hbm, wbv)

    def fire_gather(j, b):
        pltpu.async_copy(h_hbm.at[srcv.at[j]], rowsv.at[b], gsem.at[b])

    def wait_gather(j, b):
        pltpu.make_async_copy(h_hbm.at[srcv.at[j]], rowsv.at[b],
                              gsem.at[b]).wait()

    def fire_scatter(j, b):
        pltpu.async_copy(rowsv.at[b], acc.at[dstv.at[j]], ssem.at[b],
                         add=True)

    def wait_scatter(j, b):
        pltpu.make_async_copy(rowsv.at[b], acc.at[dstv.at[j]],
                              ssem.at[b]).wait()

    we = [wbv[0, pl.ds(g * 16, 16)] for g in range(4)]
    be = [wbv[1, pl.ds(g * 16, 16)] for g in range(4)]

    def run(start, cpw):
        with jax.named_scope("stage_idx"):
            pltpu.sync_copy(src_hbm.at[pl.ds(start, cpw)],
                            srcv.at[pl.ds(0, cpw)])
            pltpu.sync_copy(dst_hbm.at[pl.ds(start, cpw)],
                            dstv.at[pl.ds(0, cpw)])
            pltpu.sync_copy(ea_hbm.at[pl.ds(start, cpw)],
                            eav.at[pl.ds(0, cpw)])

        for b in range(LOOK):
            fire_gather(b, b)

        with jax.named_scope("pre_barrier"):
            plsc.subcore_barrier()

        @pl.loop(0, cpw // NBUF)
        def _outer(jo):
            for b in range(NBUF):
                j = jo * NBUF + b
                bn = (b + LOOK) % NBUF

                @pl.when(j + LOOK < cpw)
                def _fire():
                    @pl.when(j + LOOK >= NBUF)
                    def _drain():
                        wait_scatter(j + LOOK - NBUF, bn)
                    fire_gather(j + LOOK, bn)

                wait_gather(j, b)

                @pl.loop(0, C // 16)
                def _blk(jj):
                    a_vec = eav[j, pl.ds(jj * 16, 16)]
                    for ii in range(16):
                        i = jj * 16 + ii
                        a = a_vec[ii]
                        for g in range(4):
                            sl = pl.ds(g * 16, 16)
                            e = we[g] * a + be[g]
                            rowsv[b, i, sl] = jnp.maximum(
                                rowsv[b, i, sl] + e, 0.0)

                fire_scatter(j, b)

        with jax.named_scope("drain"):
            for b in range(NBUF):
                wait_scatter(cpw - NBUF + b, b)

    @pl.when(c == 0)
    def _core0():
        run(s * CPW0, CPW0)

    @pl.when(c == 1)
    def _core1():
        run(16 * CPW0 + s * CPW1, CPW1)

    with jax.named_scope("post_barrier"):
        plsc.subcore_barrier()
    with jax.named_scope("writeout"):
        pltpu.sync_copy(acc.at[pl.ds(s * SUB_ROWS, SUB_ROWS)],
                        out_hbm.at[c, pl.ds(s * SUB_ROWS, SUB_ROWS)])


@functools.lru_cache(maxsize=None)
def _mp_call():
    return pl.kernel(
        _mp_body,
        mesh=plsc.VectorSubcoreMesh(core_axis_name="c", subcore_axis_name="s"),
        out_type=jax.ShapeDtypeStruct((2, N_PAD, H), jnp.float32),
        scratch_types=[
            pltpu.VMEM((CPWMAX, C), jnp.int32),
            pltpu.VMEM((CPWMAX, C), jnp.int32),
            pltpu.VMEM((CPWMAX, C), jnp.float32),
            pltpu.VMEM((NBUF, C, H), jnp.float32),
            pltpu.VMEM((2, H), jnp.float32),
            pltpu.VMEM_SHARED((N_PAD, H), jnp.float32),
            pltpu.SemaphoreType.DMA((NBUF,)),
            pltpu.SemaphoreType.DMA((NBUF,)),
        ],
        compiler_params=pltpu.CompilerParams(use_tc_tiling_on_sc=False),
    )


def _mlp_body(first, last, h_ref, a0_ref, a1_ref, eps_ref, wm1, bm1, gm, bt,
              wm2, bm2, wa, ba, out_ref):
    h = h_ref[...]
    z = h * (1.0 + eps_ref[0, 0]) + a0_ref[...] + a1_ref[...]
    y = jnp.dot(z, wm1[...], preferred_element_type=jnp.float32) + bm1[...]
    mean = jnp.mean(y, axis=0, keepdims=True)
    var = jnp.mean(jnp.square(y - mean), axis=0, keepdims=True)
    y = (y - mean) / jnp.sqrt(var + 1e-5) * gm[...] + bt[...]
    y = jnp.maximum(y, 0.0)
    cc = jnp.dot(y, wm2[...], preferred_element_type=jnp.float32) + bm2[...]
    hn = jnp.maximum(cc, 0.0) if first else h + jnp.maximum(cc, 0.0)
    if last:
        o = jnp.dot(hn, wa[...], preferred_element_type=jnp.float32) + ba[...]
        sp = jnp.maximum(o, 0.0) + jnp.log1p(jnp.exp(-jnp.abs(o)))
        col = lax.broadcasted_iota(jnp.int32, o.shape, 1)
        out_ref[...] = jnp.where(col == 0, o, sp)
    else:
        out_ref[...] = hn


def _mlp(first, last, h, a0, a1, p, Wa, ba):
    odim = 2 if last else H
    body = functools.partial(_mlp_body, first, last)
    return pl.pallas_call(
        body,
        out_shape=jax.ShapeDtypeStruct((N, odim), jnp.float32),
    )(h, a0, a1, p['eps'].reshape(1, 1),
      p['Wm1'], p['bm1'].reshape(1, H),
      p['gamma'].reshape(1, H), p['beta'].reshape(1, H),
      p['Wm2'], p['bm2'].reshape(1, H),
      Wa, ba.reshape(1, 2))


def kernel(ensemble, x, edge_index, edge_attr, deepset, Wd, bd, convs, Wa, ba):
    nf = _pre(ensemble, x, deepset, Wd, bd)

    src = jnp.concatenate(
        [edge_index[0], jnp.zeros((E_PAD - E,), jnp.int32)]
    ).reshape(TOTCH, C)
    dst3 = jnp.concatenate(
        [edge_index[1], jnp.full((E_PAD - E,), N_PAD - 1, jnp.int32)]
    ).reshape(TOTCH, C)
    ea = jnp.concatenate(
        [edge_attr[:, 0], jnp.zeros((E_PAD - E,), jnp.float32)]
    ).reshape(TOTCH, C)
    zeros = jnp.zeros((N_PAD, H), jnp.float32)

    h = nf
    for i, p in enumerate(convs):
        wb = jnp.stack([p['We'][0], p['be']])
        out = _mp_call()(h, src, dst3, ea, wb, zeros)
        h = _mlp(i == 0, i == len(convs) - 1,
                 h, out[0, :N], out[1, :N], p, Wa, ba)
    return h


# speedup vs baseline: 9.2328x; 1.8905x over previous
import functools

import jax
import jax.numpy as jnp
from jax import lax
from jax.experimental import pallas as pl
from jax.experimental.pallas import tpu as pltpu
from jax.experimental.pallas import tpu_sc as plsc

N = 10000
E = 320000
D_IN = 128
H = 64
ENS = 10

NB = 200
GRID_PRE = N // NB

C = 128
NW = 32
TOTCH = 2560
E_PAD = TOTCH * C
CPW0 = 80
CPW1 = 80
CPWMAX = max(CPW0, CPW1)
NBUF = 4
LOOK = 2
SUB_ROWS = 648
N_PAD = SUB_ROWS * 16


def _pre_body(ens_ref, x_ref, w1, b1, w2, b2, w3, b3, w4, b4, wdx, wde, bd,
              out_ref):
    ens = ens_ref[...].reshape(NB * ENS, D_IN)
    phi = jnp.maximum(jnp.dot(ens, w1[...], preferred_element_type=jnp.float32)
                      + b1[...], 0.0)
    phi = jnp.dot(phi, w2[...], preferred_element_type=jnp.float32) + b2[...]
    agg = phi.reshape(NB, ENS, H).sum(axis=1)
    emb = jnp.maximum(jnp.dot(agg, w3[...], preferred_element_type=jnp.float32)
                      + b3[...], 0.0)
    emb = jnp.dot(emb, w4[...], preferred_element_type=jnp.float32) + b4[...]
    nf = (jnp.dot(x_ref[...], wdx[...], preferred_element_type=jnp.float32)
          + jnp.dot(emb, wde[...], preferred_element_type=jnp.float32)
          + bd[...])
    out_ref[...] = nf


def _pre(ensemble, x, ds, Wd, bd):
    full = lambda shape: pl.BlockSpec(shape, lambda i: (0,) * len(shape))
    return pl.pallas_call(
        _pre_body,
        grid=(GRID_PRE,),
        in_specs=[
            pl.BlockSpec((NB, ENS, D_IN), lambda i: (i, 0, 0)),
            pl.BlockSpec((NB, D_IN), lambda i: (i, 0)),
            full((D_IN, H)), full((1, H)),
            full((H, H)), full((1, H)),
            full((H, H)), full((1, H)),
            full((H, H)), full((1, H)),
            full((D_IN, H)), full((H, H)), full((1, H)),
        ],
        out_specs=pl.BlockSpec((NB, H), lambda i: (i, 0)),
        out_shape=jax.ShapeDtypeStruct((N, H), jnp.float32),
    )(ensemble, x,
      ds['W1'], ds['b1'].reshape(1, H),
      ds['W2'], ds['b2'].reshape(1, H),
      ds['W3'], ds['b3'].reshape(1, H),
      ds['W4'], ds['b4'].reshape(1, H),
      Wd[:D_IN], Wd[D_IN:], bd.reshape(1, H))


def _mp_body(h_hbm, src_hbm, dst_hbm, ea_hbm, wb_hbm, z_hbm, out_hbm,
             srcv, dstv, eav, rowsv, wbv, acc, gsem, ssem):
    c = lax.axis_index("c")
    s = lax.axis_index("s")

    with jax.named_scope("zero_acc"):
        pltpu.sync_copy(z_hbm.at[pl.ds(s * SUB_ROWS, SUB_ROWS)],
                        acc.at[pl.ds(s * SUB_ROWS, SUB_ROWS)])
        pltpu.sync_copy(wb_hbm, wbv)

    def fire_gather(j, b):
        pltpu.async_copy(h_hbm.at[srcv.at[j]], rowsv.at[b], gsem.at[b])

    def wait_gather(j, b):
        pltpu.make_async_copy(h_hbm.at[srcv.at[j]], rowsv.at[b],
                              gsem.at[b]).wait()

    def fire_scatter(j, b):
        pltpu.async_copy(rowsv.at[b], acc.at[dstv.at[j]], ssem.at[b],
                         add=True)

    def wait_scatter(j, b):
        pltpu.make_async_copy(rowsv.at[b], acc.at[dstv.at[j]],
                              ssem.at[b]).wait()

    we = [wbv[0, pl.ds(g * 16, 16)] for g in range(4)]
    be = [wbv[1, pl.ds(g * 16, 16)] for g in range(4)]

    def run(start, cpw):
        with jax.named_scope("stage_idx"):
            pltpu.sync_copy(src_hbm.at[pl.ds(start, cpw)],
                            srcv.at[pl.ds(0, cpw)])
            pltpu.sync_copy(dst_hbm.at[pl.ds(start, cpw)],
                            dstv.at[pl.ds(0, cpw)])
            pltpu.sync_copy(ea_hbm.at[pl.ds(start, cpw)],
                            eav.at[pl.ds(0, cpw)])

        for b in range(LOOK):
            fire_gather(b, b)

        with jax.named_scope("pre_barrier"):
            plsc.subcore_barrier()

        @pl.loop(0, cpw // NBUF)
        def _outer(jo):
            for b in range(NBUF):
                j = jo * NBUF + b
                bn = (b + LOOK) % NBUF

                @pl.when(j + LOOK < cpw)
                def _fire():
                    @pl.when(j + LOOK >= NBUF)
                    def _drain():
                        wait_scatter(j + LOOK - NBUF, bn)
                    fire_gather(j + LOOK, bn)

                wait_gather(j, b)

                @pl.loop(0, C // 16)
                def _blk(jj):
                    a_vec = eav[j, pl.ds(jj * 16, 16)]
                    for ii in range(16):
                        i = jj * 16 + ii
                        a = a_vec[ii]
                        for g in range(4):
                            sl = pl.ds(g * 16, 16)
                            e = we[g] * a + be[g]
                            rowsv[b, i, sl] = jnp.maximum(
                                rowsv[b, i, sl] + e, 0.0)

                fire_scatter(j, b)

        with jax.named_scope("drain"):
            for b in range(NBUF):
                wait_scatter(cpw - NBUF + b, b)

    @pl.when(c == 0)
    def _core0():
        run(s * CPW0, CPW0)

    @pl.when(c == 1)
    def _core1():
        run(16 * CPW0 + s * CPW1, CPW1)

    with jax.named_scope("post_barrier"):
        plsc.subcore_barrier()
    with jax.named_scope("writeout"):
        pltpu.sync_copy(acc.at[pl.ds(s * SUB_ROWS, SUB_ROWS)],
                        out_hbm.at[c, pl.ds(s * SUB_ROWS, SUB_ROWS)])


@functools.lru_cache(maxsize=None)
def _mp_call():
    return pl.kernel(
        _mp_body,
        mesh=plsc.VectorSubcoreMesh(core_axis_name="c", subcore_axis_name="s"),
        out_type=jax.ShapeDtypeStruct((2, N_PAD, H), jnp.float32),
        scratch_types=[
            pltpu.VMEM((CPWMAX, C), jnp.int32),
            pltpu.VMEM((CPWMAX, C), jnp.int32),
            pltpu.VMEM((CPWMAX, C), jnp.float32),
            pltpu.VMEM((NBUF, C, H), jnp.float32),
            pltpu.VMEM((2, H), jnp.float32),
            pltpu.VMEM_SHARED((N_PAD, H), jnp.float32),
            pltpu.SemaphoreType.DMA((NBUF,)),
            pltpu.SemaphoreType.DMA((NBUF,)),
        ],
        compiler_params=pltpu.CompilerParams(use_tc_tiling_on_sc=False),
    )


def _mlp_body(first, last, h_ref, a0_ref, a1_ref, eps_ref, wm1, bm1, gm, bt,
              wm2, bm2, wa, ba, out_ref):
    h = h_ref[...]
    z = h * (1.0 + eps_ref[0, 0]) + a0_ref[...] + a1_ref[...]
    y = jnp.dot(z, wm1[...], preferred_element_type=jnp.float32) + bm1[...]
    mean = jnp.mean(y, axis=0, keepdims=True)
    var = jnp.mean(jnp.square(y - mean), axis=0, keepdims=True)
    y = (y - mean) / jnp.sqrt(var + 1e-5) * gm[...] + bt[...]
    y = jnp.maximum(y, 0.0)
    cc = jnp.dot(y, wm2[...], preferred_element_type=jnp.float32) + bm2[...]
    hn = jnp.maximum(cc, 0.0) if first else h + jnp.maximum(cc, 0.0)
    if last:
        o = jnp.dot(hn, wa[...], preferred_element_type=jnp.float32) + ba[...]
        sp = jnp.maximum(o, 0.0) + jnp.log1p(jnp.exp(-jnp.abs(o)))
        col = lax.broadcasted_iota(jnp.int32, o.shape, 1)
        out_ref[...] = jnp.where(col == 0, o, sp)
    else:
        out_ref[...] = hn


def _mlp(first, last, h, a0, a1, p, Wa, ba):
    odim = 2 if last else H
    body = functools.partial(_mlp_body, first, last)
    return pl.pallas_call(
        body,
        out_shape=jax.ShapeDtypeStruct((N, odim), jnp.float32),
    )(h, a0, a1, p['eps'].reshape(1, 1),
      p['Wm1'], p['bm1'].reshape(1, H),
      p['gamma'].reshape(1, H), p['beta'].reshape(1, H),
      p['Wm2'], p['bm2'].reshape(1, H),
      Wa, ba.reshape(1, 2))


def kernel(ensemble, x, edge_index, edge_attr, deepset, Wd, bd, convs, Wa, ba):
    nf = _pre(ensemble, x, deepset, Wd, bd)

    pad_i = jnp.arange(E_PAD - E, dtype=jnp.int32)
    src = jnp.concatenate(
        [edge_index[0], pad_i % N]
    ).reshape(TOTCH, C)
    dst3 = jnp.concatenate(
        [edge_index[1], N + (pad_i % 128)]
    ).reshape(TOTCH, C)
    ea = jnp.concatenate(
        [edge_attr[:, 0], jnp.zeros((E_PAD - E,), jnp.float32)]
    ).reshape(TOTCH, C)
    zeros = jnp.zeros((N_PAD, H), jnp.float32)

    h = nf
    for i, p in enumerate(convs):
        wb = jnp.stack([p['We'][0], p['be']])
        out = _mp_call()(h, src, dst3, ea, wb, zeros)
        h = _mlp(i == 0, i == len(convs) - 1,
                 h, out[0, :N], out[1, :N], p, Wa, ba)
    return h


# speedup vs baseline: 9.3193x; 1.0094x over previous
import functools

import jax
import jax.numpy as jnp
from jax import lax
from jax.experimental import pallas as pl
from jax.experimental.pallas import tpu as pltpu
from jax.experimental.pallas import tpu_sc as plsc

N = 10000
E = 320000
D_IN = 128
H = 64
ENS = 10

NB = 200
GRID_PRE = N // NB

C = 128
NW = 32
TOTCH = 2560
E_PAD = TOTCH * C
CPW0 = 80
CPW1 = 80
CPWMAX = max(CPW0, CPW1)
NBUF = 4
LOOK = 2
SUB_ROWS = 648
N_PAD = SUB_ROWS * 16


def _pre_body(ens_ref, x_ref, w1, b1, w2, b2, w3, b3, w4, b4, wdx, wde, bd,
              out_ref):
    ens = ens_ref[...]
    phi = jnp.maximum(jnp.dot(ens, w1[...], preferred_element_type=jnp.float32)
                      + b1[...], 0.0)
    phi = jnp.dot(phi.astype(jnp.bfloat16), w2[...],
                  preferred_element_type=jnp.float32) + b2[...]
    agg = phi.reshape(NB, ENS, H).sum(axis=1)
    emb = jnp.maximum(jnp.dot(agg, w3[...], preferred_element_type=jnp.float32)
                      + b3[...], 0.0)
    emb = jnp.dot(emb, w4[...], preferred_element_type=jnp.float32) + b4[...]
    nf = (jnp.dot(x_ref[...], wdx[...], preferred_element_type=jnp.float32)
          + jnp.dot(emb, wde[...], preferred_element_type=jnp.float32)
          + bd[...])
    out_ref[...] = nf


def _pre(ensemble, x, ds, Wd, bd):
    full = lambda shape: pl.BlockSpec(shape, lambda i: (0,) * len(shape))
    return pl.pallas_call(
        _pre_body,
        grid=(GRID_PRE,),
        in_specs=[
            pl.BlockSpec((NB * ENS, D_IN), lambda i: (i, 0)),
            pl.BlockSpec((NB, D_IN), lambda i: (i, 0)),
            full((D_IN, H)), full((1, H)),
            full((H, H)), full((1, H)),
            full((H, H)), full((1, H)),
            full((H, H)), full((1, H)),
            full((D_IN, H)), full((H, H)), full((1, H)),
        ],
        out_specs=pl.BlockSpec((NB, H), lambda i: (i, 0)),
        out_shape=jax.ShapeDtypeStruct((N, H), jnp.float32),
    )(ensemble.reshape(N * ENS, D_IN).astype(jnp.bfloat16), x,
      ds['W1'].astype(jnp.bfloat16), ds['b1'].reshape(1, H),
      ds['W2'].astype(jnp.bfloat16), ds['b2'].reshape(1, H),
      ds['W3'], ds['b3'].reshape(1, H),
      ds['W4'], ds['b4'].reshape(1, H),
      Wd[:D_IN], Wd[D_IN:], bd.reshape(1, H))


def _mp_body(h_hbm, src_hbm, dst_hbm, ea_hbm, wb_hbm, z_hbm, out_hbm,
             srcv, dstv, eav, rowsv, wbv, acc, gsem, ssem):
    c = lax.axis_index("c")
    s = lax.axis_index("s")

    with jax.named_scope("zero_acc"):
        pltpu.sync_copy(z_hbm.at[pl.ds(s * SUB_ROWS, SUB_ROWS)],
                        acc.at[pl.ds(s * SUB_ROWS, SUB_ROWS)])
        pltpu.sync_copy(wb_hbm, wbv)

    def fire_gather(j, b):
        pltpu.async_copy(h_hbm.at[srcv.at[j]], rowsv.at[b], gsem.at[b])

    def wait_gather(j, b):
        pltpu.make_async_copy(h_hbm.at[srcv.at[j]], rowsv.at[b],
                              gsem.at[b]).wait()

    def fire_scatter(j, b):
        pltpu.async_copy(rowsv.at[b], acc.at[dstv.at[j]], ssem.at[b],
                         add=True)

    def wait_scatter(j, b):
        pltpu.make_async_copy(rowsv.at[b], acc.at[dstv.at[j]],
                              ssem.at[b]).wait()

    we = [wbv[0, pl.ds(g * 16, 16)] for g in range(4)]
    be = [wbv[1, pl.ds(g * 16, 16)] for g in range(4)]

    def run(start, cpw):
        with jax.named_scope("stage_idx"):
            pltpu.sync_copy(src_hbm.at[pl.ds(start, cpw)],
                            srcv.at[pl.ds(0, cpw)])
            pltpu.sync_copy(dst_hbm.at[pl.ds(start, cpw)],
                            dstv.at[pl.ds(0, cpw)])
            pltpu.sync_copy(ea_hbm.at[pl.ds(start, cpw)],
                            eav.at[pl.ds(0, cpw)])

        for b in range(LOOK):
            fire_gather(b, b)

        with jax.named_scope("pre_barrier"):
            plsc.subcore_barrier()

        @pl.loop(0, cpw // NBUF)
        def _outer(jo):
            for b in range(NBUF):
                j = jo * NBUF + b
                bn = (b + LOOK) % NBUF

                @pl.when(j + LOOK < cpw)
                def _fire():
                    @pl.when(j + LOOK >= NBUF)
                    def _drain():
                        wait_scatter(j + LOOK - NBUF, bn)
                    fire_gather(j + LOOK, bn)

                wait_gather(j, b)

                @pl.loop(0, C // 16)
                def _blk(jj):
                    a_vec = eav[j, pl.ds(jj * 16, 16)]
                    for ii in range(16):
                        i = jj * 16 + ii
                        a = a_vec[ii]
                        for g in range(4):
                            sl = pl.ds(g * 16, 16)
                            e = we[g] * a + be[g]
                            rowsv[b, i, sl] = jnp.maximum(
                                rowsv[b, i, sl] + e, 0.0)

                fire_scatter(j, b)

        with jax.named_scope("drain"):
            for b in range(NBUF):
                wait_scatter(cpw - NBUF + b, b)

    @pl.when(c == 0)
    def _core0():
        run(s * CPW0, CPW0)

    @pl.when(c == 1)
    def _core1():
        run(16 * CPW0 + s * CPW1, CPW1)

    with jax.named_scope("post_barrier"):
        plsc.subcore_barrier()
    with jax.named_scope("writeout"):
        pltpu.sync_copy(acc.at[pl.ds(s * SUB_ROWS, SUB_ROWS)],
                        out_hbm.at[c, pl.ds(s * SUB_ROWS, SUB_ROWS)])


@functools.lru_cache(maxsize=None)
def _mp_call():
    return pl.kernel(
        _mp_body,
        mesh=plsc.VectorSubcoreMesh(core_axis_name="c", subcore_axis_name="s"),
        out_type=jax.ShapeDtypeStruct((2, N_PAD, H), jnp.float32),
        scratch_types=[
            pltpu.VMEM((CPWMAX, C), jnp.int32),
            pltpu.VMEM((CPWMAX, C), jnp.int32),
            pltpu.VMEM((CPWMAX, C), jnp.float32),
            pltpu.VMEM((NBUF, C, H), jnp.float32),
            pltpu.VMEM((2, H), jnp.float32),
            pltpu.VMEM_SHARED((N_PAD, H), jnp.float32),
            pltpu.SemaphoreType.DMA((NBUF,)),
            pltpu.SemaphoreType.DMA((NBUF,)),
        ],
        compiler_params=pltpu.CompilerParams(use_tc_tiling_on_sc=False),
    )


def _mlp_body(first, last, h_ref, agg_ref, eps_ref, wm1, bm1, gm, bt,
              wm2, bm2, wa, ba, out_ref):
    h = h_ref[...]
    z = h * (1.0 + eps_ref[0, 0]) + agg_ref[0, :N, :] + agg_ref[1, :N, :]
    y = jnp.dot(z, wm1[...], preferred_element_type=jnp.float32) + bm1[...]
    mean = jnp.mean(y, axis=0, keepdims=True)
    var = jnp.mean(jnp.square(y - mean), axis=0, keepdims=True)
    y = (y - mean) / jnp.sqrt(var + 1e-5) * gm[...] + bt[...]
    y = jnp.maximum(y, 0.0)
    cc = jnp.dot(y, wm2[...], preferred_element_type=jnp.float32) + bm2[...]
    hn = jnp.maximum(cc, 0.0) if first else h + jnp.maximum(cc, 0.0)
    if last:
        o = jnp.dot(hn, wa[...], preferred_element_type=jnp.float32) + ba[...]
        sp = jnp.maximum(o, 0.0) + jnp.log1p(jnp.exp(-jnp.abs(o)))
        col = lax.broadcasted_iota(jnp.int32, o.shape, 1)
        out_ref[...] = jnp.where(col == 0, o, sp)
    else:
        out_ref[...] = hn


def _mlp(first, last, h, agg, p, Wa, ba):
    odim = 2 if last else H
    body = functools.partial(_mlp_body, first, last)
    return pl.pallas_call(
        body,
        out_shape=jax.ShapeDtypeStruct((N, odim), jnp.float32),
    )(h, agg, p['eps'].reshape(1, 1),
      p['Wm1'], p['bm1'].reshape(1, H),
      p['gamma'].reshape(1, H), p['beta'].reshape(1, H),
      p['Wm2'], p['bm2'].reshape(1, H),
      Wa, ba.reshape(1, 2))


def kernel(ensemble, x, edge_index, edge_attr, deepset, Wd, bd, convs, Wa, ba):
    nf = _pre(ensemble, x, deepset, Wd, bd)

    pad_i = jnp.arange(E_PAD - E, dtype=jnp.int32)
    src = jnp.concatenate(
        [edge_index[0], pad_i % N]
    ).reshape(TOTCH, C)
    dst3 = jnp.concatenate(
        [edge_index[1], N + (pad_i % 128)]
    ).reshape(TOTCH, C)
    ea = jnp.concatenate(
        [edge_attr[:, 0], jnp.zeros((E_PAD - E,), jnp.float32)]
    ).reshape(TOTCH, C)
    zeros = jnp.zeros((N_PAD, H), jnp.float32)

    h = nf
    for i, p in enumerate(convs):
        wb = jnp.stack([p['We'][0], p['be']])
        out = _mp_call()(h, src, dst3, ea, wb, zeros)
        h = _mlp(i == 0, i == len(convs) - 1, h, out, p, Wa, ba)
    return h


# speedup vs baseline: 9.3366x; 1.0019x over previous
import functools

import jax
import jax.numpy as jnp
from jax import lax
from jax.experimental import pallas as pl
from jax.experimental.pallas import tpu as pltpu
from jax.experimental.pallas import tpu_sc as plsc

N = 10000
E = 320000
D_IN = 128
H = 64
ENS = 10

NB = 1000
GRID_PRE = N // NB

C = 128
NW = 32
TOTCH = 2560
E_PAD = TOTCH * C
CPW0 = 80
CPW1 = 80
CPWMAX = max(CPW0, CPW1)
NBUF = 4
LOOK = 2
SUB_ROWS = 648
N_PAD = SUB_ROWS * 16


def _pre_body(ens_ref, x_ref, w1, b1, w2, b2, w3, b3, w4, b4, wdx, wde, bd,
              out_ref):
    sacc = jnp.zeros((NB, H), jnp.float32)
    for e in range(ENS):
        m = ens_ref[:, e, :]
        sacc = sacc + jnp.maximum(
            jnp.dot(m, w1[...], preferred_element_type=jnp.float32) + b1[...],
            0.0)
    agg = (jnp.dot(sacc.astype(jnp.bfloat16), w2[...],
                   preferred_element_type=jnp.float32)
           + float(ENS) * b2[...])
    emb = jnp.maximum(jnp.dot(agg, w3[...], preferred_element_type=jnp.float32)
                      + b3[...], 0.0)
    emb = jnp.dot(emb, w4[...], preferred_element_type=jnp.float32) + b4[...]
    nf = (jnp.dot(x_ref[...], wdx[...], preferred_element_type=jnp.float32)
          + jnp.dot(emb, wde[...], preferred_element_type=jnp.float32)
          + bd[...])
    out_ref[...] = nf


def _pre(ensemble, x, ds, Wd, bd):
    full = lambda shape: pl.BlockSpec(shape, lambda i: (0,) * len(shape))
    return pl.pallas_call(
        _pre_body,
        grid=(GRID_PRE,),
        in_specs=[
            pl.BlockSpec((NB, ENS, D_IN), lambda i: (i, 0, 0)),
            pl.BlockSpec((NB, D_IN), lambda i: (i, 0)),
            full((D_IN, H)), full((1, H)),
            full((H, H)), full((1, H)),
            full((H, H)), full((1, H)),
            full((H, H)), full((1, H)),
            full((D_IN, H)), full((H, H)), full((1, H)),
        ],
        out_specs=pl.BlockSpec((NB, H), lambda i: (i, 0)),
        out_shape=jax.ShapeDtypeStruct((N, H), jnp.float32),
    )(ensemble.astype(jnp.bfloat16), x,
      ds['W1'].astype(jnp.bfloat16), ds['b1'].reshape(1, H),
      ds['W2'].astype(jnp.bfloat16), ds['b2'].reshape(1, H),
      ds['W3'], ds['b3'].reshape(1, H),
      ds['W4'], ds['b4'].reshape(1, H),
      Wd[:D_IN], Wd[D_IN:], bd.reshape(1, H))


def _mp_body(h_hbm, src_hbm, dst_hbm, ea_hbm, wb_hbm, z_hbm, out_hbm,
             srcv, dstv, eav, rowsv, wbv, acc, gsem, ssem):
    c = lax.axis_index("c")
    s = lax.axis_index("s")

    with jax.named_scope("zero_acc"):
        pltpu.sync_copy(z_hbm.at[pl.ds(s * SUB_ROWS, SUB_ROWS)],
                        acc.at[pl.ds(s * SUB_ROWS, SUB_ROWS)])
        pltpu.sync_copy(wb_hbm, wbv)

    def fire_gather(j, b):
        pltpu.async_copy(h_hbm.at[srcv.at[j]], rowsv.at[b], gsem.at[b])

    def wait_gather(j, b):
        pltpu.make_async_copy(h_hbm.at[srcv.at[j]], rowsv.at[b],
                              gsem.at[b]).wait()

    def fire_scatter(j, b):
        pltpu.async_copy(rowsv.at[b], acc.at[dstv.at[j]], ssem.at[b],
                         add=True)

    def wait_scatter(j, b):
        pltpu.make_async_copy(rowsv.at[b], acc.at[dstv.at[j]],
                              ssem.at[b]).wait()

    we = [wbv[0, pl.ds(g * 16, 16)] for g in range(4)]
    be = [wbv[1, pl.ds(g * 16, 16)] for g in range(4)]

    def run(start, cpw):
        with jax.named_scope("stage_idx"):
            pltpu.sync_copy(src_hbm.at[pl.ds(start, cpw)],
                            srcv.at[pl.ds(0, cpw)])
            pltpu.sync_copy(dst_hbm.at[pl.ds(start, cpw)],
                            dstv.at[pl.ds(0, cpw)])
            pltpu.sync_copy(ea_hbm.at[pl.ds(start, cpw)],
                            eav.at[pl.ds(0, cpw)])

        for b in range(LOOK):
            fire_gather(b, b)

        with jax.named_scope("pre_barrier"):
            plsc.subcore_barrier()

        @pl.loop(0, cpw // NBUF)
        def _outer(jo):
            for b in range(NBUF):
                j = jo * NBUF + b
                bn = (b + LOOK) % NBUF

                @pl.when(j + LOOK < cpw)
                def _fire():
                    @pl.when(j + LOOK >= NBUF)
                    def _drain():
                        wait_scatter(j + LOOK - NBUF, bn)
                    fire_gather(j + LOOK, bn)

                wait_gather(j, b)

                @pl.loop(0, C // 16)
                def _blk(jj):
                    a_vec = eav[j, pl.ds(jj * 16, 16)]
                    for ii in range(16):
                        i = jj * 16 + ii
                        a = a_vec[ii]
                        for g in range(4):
                            sl = pl.ds(g * 16, 16)
                            e = we[g] * a + be[g]
                            rowsv[b, i, sl] = jnp.maximum(
                                rowsv[b, i, sl] + e, 0.0)

                fire_scatter(j, b)

        with jax.named_scope("drain"):
            for b in range(NBUF):
                wait_scatter(cpw - NBUF + b, b)

    @pl.when(c == 0)
    def _core0():
        run(s * CPW0, CPW0)

    @pl.when(c == 1)
    def _core1():
        run(16 * CPW0 + s * CPW1, CPW1)

    with jax.named_scope("post_barrier"):
        plsc.subcore_barrier()
    with jax.named_scope("writeout"):
        pltpu.sync_copy(acc.at[pl.ds(s * SUB_ROWS, SUB_ROWS)],
                        out_hbm.at[c, pl.ds(s * SUB_ROWS, SUB_ROWS)])


@functools.lru_cache(maxsize=None)
def _mp_call():
    return pl.kernel(
        _mp_body,
        mesh=plsc.VectorSubcoreMesh(core_axis_name="c", subcore_axis_name="s"),
        out_type=jax.ShapeDtypeStruct((2, N_PAD, H), jnp.float32),
        scratch_types=[
            pltpu.VMEM((CPWMAX, C), jnp.int32),
            pltpu.VMEM((CPWMAX, C), jnp.int32),
            pltpu.VMEM((CPWMAX, C), jnp.float32),
            pltpu.VMEM((NBUF, C, H), jnp.float32),
            pltpu.VMEM((2, H), jnp.float32),
            pltpu.VMEM_SHARED((N_PAD, H), jnp.float32),
            pltpu.SemaphoreType.DMA((NBUF,)),
            pltpu.SemaphoreType.DMA((NBUF,)),
        ],
        compiler_params=pltpu.CompilerParams(use_tc_tiling_on_sc=False),
    )


def _mlp_body(first, last, h_ref, agg_ref, eps_ref, wm1, bm1, gm, bt,
              wm2, bm2, wa, ba, out_ref):
    h = h_ref[...]
    z = h * (1.0 + eps_ref[0, 0]) + agg_ref[0, :N, :] + agg_ref[1, :N, :]
    y = jnp.dot(z, wm1[...], preferred_element_type=jnp.float32) + bm1[...]
    mean = jnp.mean(y, axis=0, keepdims=True)
    var = jnp.mean(jnp.square(y - mean), axis=0, keepdims=True)
    y = (y - mean) / jnp.sqrt(var + 1e-5) * gm[...] + bt[...]
    y = jnp.maximum(y, 0.0)
    cc = jnp.dot(y, wm2[...], preferred_element_type=jnp.float32) + bm2[...]
    hn = jnp.maximum(cc, 0.0) if first else h + jnp.maximum(cc, 0.0)
    if last:
        o = jnp.dot(hn, wa[...], preferred_element_type=jnp.float32) + ba[...]
        sp = jnp.maximum(o, 0.0) + jnp.log1p(jnp.exp(-jnp.abs(o)))
        col = lax.broadcasted_iota(jnp.int32, o.shape, 1)
        out_ref[...] = jnp.where(col == 0, o, sp)
    else:
        out_ref[...] = hn


def _mlp(first, last, h, agg, p, Wa, ba):
    odim = 2 if last else H
    body = functools.partial(_mlp_body, first, last)
    return pl.pallas_call(
        body,
        out_shape=jax.ShapeDtypeStruct((N, odim), jnp.float32),
    )(h, agg, p['eps'].reshape(1, 1),
      p['Wm1'], p['bm1'].reshape(1, H),
      p['gamma'].reshape(1, H), p['beta'].reshape(1, H),
      p['Wm2'], p['bm2'].reshape(1, H),
      Wa, ba.reshape(1, 2))


def kernel(ensemble, x, edge_index, edge_attr, deepset, Wd, bd, convs, Wa, ba):
    nf = _pre(ensemble, x, deepset, Wd, bd)

    pad_i = jnp.arange(E_PAD - E, dtype=jnp.int32)
    src = jnp.concatenate(
        [edge_index[0], pad_i % N]
    ).reshape(TOTCH, C)
    dst3 = jnp.concatenate(
        [edge_index[1], N + (pad_i % 128)]
    ).reshape(TOTCH, C)
    ea = jnp.concatenate(
        [edge_attr[:, 0], jnp.zeros((E_PAD - E,), jnp.float32)]
    ).reshape(TOTCH, C)
    zeros = jnp.zeros((N_PAD, H), jnp.float32)

    h = nf
    for i, p in enumerate(convs):
        wb = jnp.stack([p['We'][0], p['be']])
        out = _mp_call()(h, src, dst3, ea, wb, zeros)
        h = _mlp(i == 0, i == len(convs) - 1, h, out, p, Wa, ba)
    return h


# speedup vs baseline: 12.6940x; 1.3596x over previous
import functools

import jax
import jax.numpy as jnp
from jax import lax
from jax.experimental import pallas as pl
from jax.experimental.pallas import tpu as pltpu
from jax.experimental.pallas import tpu_sc as plsc

N = 10000
E = 320000
D_IN = 128
H = 64
ENS = 10

NB = 1000
GRID_PRE = N // NB

C = 128
NW = 32
TOTCH = 2560
E_PAD = TOTCH * C
CPW0 = 80
CPW1 = 80
CPWMAX = max(CPW0, CPW1)
NBUF = 4
LOOK = 2
SUB_ROWS = 648
N_PAD = SUB_ROWS * 16


def _pre_body(ens_ref, x_ref, w1, b1, w2, b2, w3, b3, w4, b4, wdx, wde, bd,
              out_ref):
    sacc = jnp.zeros((NB, H), jnp.float32)
    for e in range(ENS):
        m = ens_ref[e].astype(jnp.bfloat16)
        sacc = sacc + jnp.maximum(
            jnp.dot(m, w1[...], preferred_element_type=jnp.float32) + b1[...],
            0.0)
    agg = (jnp.dot(sacc.astype(jnp.bfloat16), w2[...],
                   preferred_element_type=jnp.float32)
           + float(ENS) * b2[...])
    emb = jnp.maximum(jnp.dot(agg, w3[...], preferred_element_type=jnp.float32)
                      + b3[...], 0.0)
    emb = jnp.dot(emb, w4[...], preferred_element_type=jnp.float32) + b4[...]
    nf = (jnp.dot(x_ref[...], wdx[...], preferred_element_type=jnp.float32)
          + jnp.dot(emb, wde[...], preferred_element_type=jnp.float32)
          + bd[...])
    out_ref[...] = nf


def _pre(ensemble, x, ds, Wd, bd):
    full = lambda shape: pl.BlockSpec(shape, lambda i: (0,) * len(shape))
    return pl.pallas_call(
        _pre_body,
        grid=(GRID_PRE,),
        in_specs=[
            pl.BlockSpec((ENS, NB, D_IN), lambda i: (0, i, 0)),
            pl.BlockSpec((NB, D_IN), lambda i: (i, 0)),
            full((D_IN, H)), full((1, H)),
            full((H, H)), full((1, H)),
            full((H, H)), full((1, H)),
            full((H, H)), full((1, H)),
            full((D_IN, H)), full((H, H)), full((1, H)),
        ],
        out_specs=pl.BlockSpec((NB, H), lambda i: (i, 0)),
        out_shape=jax.ShapeDtypeStruct((N, H), jnp.float32),
    )(ensemble.transpose(1, 0, 2), x,
      ds['W1'].astype(jnp.bfloat16), ds['b1'].reshape(1, H),
      ds['W2'].astype(jnp.bfloat16), ds['b2'].reshape(1, H),
      ds['W3'], ds['b3'].reshape(1, H),
      ds['W4'], ds['b4'].reshape(1, H),
      Wd[:D_IN], Wd[D_IN:], bd.reshape(1, H))


def _mp_body(h_hbm, src_hbm, dst_hbm, ea_hbm, wb_hbm, z_hbm, out_hbm,
             srcv, dstv, eav, rowsv, wbv, acc, gsem, ssem):
    c = lax.axis_index("c")
    s = lax.axis_index("s")

    with jax.named_scope("zero_acc"):
        pltpu.sync_copy(z_hbm.at[pl.ds(s * SUB_ROWS, SUB_ROWS)],
                        acc.at[pl.ds(s * SUB_ROWS, SUB_ROWS)])
        pltpu.sync_copy(wb_hbm, wbv)

    def fire_gather(j, b):
        pltpu.async_copy(h_hbm.at[srcv.at[j]], rowsv.at[b], gsem.at[b])

    def wait_gather(j, b):
        pltpu.make_async_copy(h_hbm.at[srcv.at[j]], rowsv.at[b],
                              gsem.at[b]).wait()

    def fire_scatter(j, b):
        pltpu.async_copy(rowsv.at[b], acc.at[dstv.at[j]], ssem.at[b],
                         add=True)

    def wait_scatter(j, b):
        pltpu.make_async_copy(rowsv.at[b], acc.at[dstv.at[j]],
                              ssem.at[b]).wait()

    we = [wbv[0, pl.ds(g * 16, 16)] for g in range(4)]
    be = [wbv[1, pl.ds(g * 16, 16)] for g in range(4)]

    def run(start, cpw):
        with jax.named_scope("stage_idx"):
            pltpu.sync_copy(src_hbm.at[pl.ds(start, cpw)],
                            srcv.at[pl.ds(0, cpw)])
            pltpu.sync_copy(dst_hbm.at[pl.ds(start, cpw)],
                            dstv.at[pl.ds(0, cpw)])
            pltpu.sync_copy(ea_hbm.at[pl.ds(start, cpw)],
                            eav.at[pl.ds(0, cpw)])

        for b in range(LOOK):
            fire_gather(b, b)

        with jax.named_scope("pre_barrier"):
            plsc.subcore_barrier()

        @pl.loop(0, cpw // NBUF)
        def _outer(jo):
            for b in range(NBUF):
                j = jo * NBUF + b
                bn = (b + LOOK) % NBUF

                @pl.when(j + LOOK < cpw)
                def _fire():
                    @pl.when(j + LOOK >= NBUF)
                    def _drain():
                        wait_scatter(j + LOOK - NBUF, bn)
                    fire_gather(j + LOOK, bn)

                wait_gather(j, b)

                @pl.loop(0, C // 16)
                def _blk(jj):
                    a_vec = eav[j, pl.ds(jj * 16, 16)]
                    for ii in range(16):
                        i = jj * 16 + ii
                        a = a_vec[ii]
                        for g in range(4):
                            sl = pl.ds(g * 16, 16)
                            e = we[g] * a + be[g]
                            rowsv[b, i, sl] = jnp.maximum(
                                rowsv[b, i, sl] + e, 0.0)

                fire_scatter(j, b)

        with jax.named_scope("drain"):
            for b in range(NBUF):
                wait_scatter(cpw - NBUF + b, b)

    @pl.when(c == 0)
    def _core0():
        run(s * CPW0, CPW0)

    @pl.when(c == 1)
    def _core1():
        run(16 * CPW0 + s * CPW1, CPW1)

    with jax.named_scope("post_barrier"):
        plsc.subcore_barrier()
    with jax.named_scope("writeout"):
        pltpu.sync_copy(acc.at[pl.ds(s * SUB_ROWS, SUB_ROWS)],
                        out_hbm.at[c, pl.ds(s * SUB_ROWS, SUB_ROWS)])


@functools.lru_cache(maxsize=None)
def _mp_call():
    return pl.kernel(
        _mp_body,
        mesh=plsc.VectorSubcoreMesh(core_axis_name="c", subcore_axis_name="s"),
        out_type=jax.ShapeDtypeStruct((2, N_PAD, H), jnp.float32),
        scratch_types=[
            pltpu.VMEM((CPWMAX, C), jnp.int32),
            pltpu.VMEM((CPWMAX, C), jnp.int32),
            pltpu.VMEM((CPWMAX, C), jnp.float32),
            pltpu.VMEM((NBUF, C, H), jnp.float32),
            pltpu.VMEM((2, H), jnp.float32),
            pltpu.VMEM_SHARED((N_PAD, H), jnp.float32),
            pltpu.SemaphoreType.DMA((NBUF,)),
            pltpu.SemaphoreType.DMA((NBUF,)),
        ],
        compiler_params=pltpu.CompilerParams(use_tc_tiling_on_sc=False),
    )


def _mlp_body(first, last, h_ref, agg_ref, eps_ref, wm1, bm1, gm, bt,
              wm2, bm2, wa, ba, out_ref):
    h = h_ref[...]
    z = h * (1.0 + eps_ref[0, 0]) + agg_ref[0, :N, :] + agg_ref[1, :N, :]
    y = jnp.dot(z, wm1[...], preferred_element_type=jnp.float32) + bm1[...]
    mean = jnp.mean(y, axis=0, keepdims=True)
    var = jnp.mean(jnp.square(y - mean), axis=0, keepdims=True)
    y = (y - mean) / jnp.sqrt(var + 1e-5) * gm[...] + bt[...]
    y = jnp.maximum(y, 0.0)
    cc = jnp.dot(y, wm2[...], preferred_element_type=jnp.float32) + bm2[...]
    hn = jnp.maximum(cc, 0.0) if first else h + jnp.maximum(cc, 0.0)
    if last:
        o = jnp.dot(hn, wa[...], preferred_element_type=jnp.float32) + ba[...]
        sp = jnp.maximum(o, 0.0) + jnp.log1p(jnp.exp(-jnp.abs(o)))
        col = lax.broadcasted_iota(jnp.int32, o.shape, 1)
        out_ref[...] = jnp.where(col == 0, o, sp)
    else:
        out_ref[...] = hn


def _mlp(first, last, h, agg, p, Wa, ba):
    odim = 2 if last else H
    body = functools.partial(_mlp_body, first, last)
    return pl.pallas_call(
        body,
        out_shape=jax.ShapeDtypeStruct((N, odim), jnp.float32),
    )(h, agg, p['eps'].reshape(1, 1),
      p['Wm1'], p['bm1'].reshape(1, H),
      p['gamma'].reshape(1, H), p['beta'].reshape(1, H),
      p['Wm2'], p['bm2'].reshape(1, H),
      Wa, ba.reshape(1, 2))


def kernel(ensemble, x, edge_index, edge_attr, deepset, Wd, bd, convs, Wa, ba):
    nf = _pre(ensemble, x, deepset, Wd, bd)

    pad_i = jnp.arange(E_PAD - E, dtype=jnp.int32)
    src = jnp.concatenate(
        [edge_index[0], pad_i % N]
    ).reshape(TOTCH, C)
    dst3 = jnp.concatenate(
        [edge_index[1], N + (pad_i % 128)]
    ).reshape(TOTCH, C)
    ea = jnp.concatenate(
        [edge_attr[:, 0], jnp.zeros((E_PAD - E,), jnp.float32)]
    ).reshape(TOTCH, C)
    zeros = jnp.zeros((N_PAD, H), jnp.float32)

    h = nf
    for i, p in enumerate(convs):
        wb = jnp.stack([p['We'][0], p['be']])
        out = _mp_call()(h, src, dst3, ea, wb, zeros)
        h = _mlp(i == 0, i == len(convs) - 1, h, out, p, Wa, ba)
    return h
